# baseline (speedup 1.0000x reference)
import functools

import jax
import jax.numpy as jnp
from jax import lax
from jax.experimental import pallas as pl
from jax.experimental.pallas import tpu as pltpu
from jax.experimental.pallas import tpu_sc as plsc

B, N, E, F, H, S, L = 8, 2048, 32768, 256, 256, 2048, 8
RB = 256
NRB = N // RB

CHUNK = 256
NCH = N // CHUNK
SPROWS = CHUNK * (N // 16)
TSL = SPROWS // 16
EPT = E // 16
NCHK = EPT // 16


def _sc_build_body(edges, a_out, deg_out, srcv, dstv, val_v, idx_v, zbuf,
                   spmem):
    c = lax.axis_index("c")
    t = lax.axis_index("s")
    SPB = CHUNK * N
    TSZ = SPB // 16
    DEG0 = 2 * SPB

    def fill(buf, n16, x):
        def w(i, carry):
            buf[pl.ds(i * 16, 16)] = jnp.full((16,), x, jnp.float32)
            return carry
        lax.fori_loop(0, n16, w, 0)

    fill(zbuf, TSZ // 16, 0.0)
    fill(val_v, EPT // 16, 1.0)

    pltpu.sync_copy(zbuf.at[pl.ds(0, 512)],
                    spmem.at[pl.ds(DEG0 + t * 512, 512)])
    plsc.subcore_barrier()
    for gl in range(B // 2):
        g = c * (B // 2) + gl
        pltpu.sync_copy(edges.at[g, 1, pl.ds(t * EPT, EPT)], dstv)

        def dchunk(k, carry):
            d16 = dstv[pl.ds(k * 16, 16)]
            idx_v[lax.shift_right_logical(k, 3),
                  pl.ds(jnp.bitwise_and(k, 7) * 16, 16)] = DEG0 + gl * N + d16
            return carry

        lax.fori_loop(0, NCHK, dchunk, 0)
        for j in range(16):
            pltpu.sync_copy(val_v.at[pl.ds(j * 128, 128)],
                            spmem.at[idx_v.at[j]], add=True)
    plsc.subcore_barrier()

    @pl.when(t < B // 2)
    def _():
        pltpu.sync_copy(spmem.at[pl.ds(DEG0 + t * N, N)],
                        deg_out.at[c * (B // 2) + t])

    pltpu.sync_copy(zbuf, spmem.at[pl.ds(t * TSZ, TSZ)])
    pltpu.sync_copy(zbuf, spmem.at[pl.ds(SPB + t * TSZ, TSZ)])
    plsc.subcore_barrier()

    passes = [(step, g) for step in range(NCH // 2) for g in range(B)]
    for p, (step, g) in enumerate(passes):
        par = p & 1
        ch = step * 2 + c
        lo = ch * CHUNK
        base = par * SPB
        pltpu.sync_copy(edges.at[g, 0, pl.ds(t * EPT, EPT)], srcv)
        pltpu.sync_copy(edges.at[g, 1, pl.ds(t * EPT, EPT)], dstv)

        def achunk(k, carry):
            s16 = srcv[pl.ds(k * 16, 16)]
            d16 = dstv[pl.ds(k * 16, 16)]
            m = (d16 >= lo) & (d16 < lo + CHUNK)
            fidx = base + (d16 - lo) * N + s16
            idx_v[lax.shift_right_logical(k, 3),
                  pl.ds(jnp.bitwise_and(k, 7) * 16, 16)] = (
                jnp.where(m, fidx, base))
            val_v[pl.ds(k * 16, 16)] = jnp.where(m, 1.0, 0.0).astype(
                jnp.float32)
            return carry

        lax.fori_loop(0, NCHK, achunk, 0)
        plsc.subcore_barrier()
        if p > 0:
            pstep, pg = passes[p - 1]
            pch = pstep * 2 + c
            pbase = (1 - par) * SPB
            pltpu.sync_copy(spmem.at[pl.ds(pbase + t * TSZ, TSZ)],
                            a_out.at[pg, pch, t])
            pltpu.sync_copy(zbuf, spmem.at[pl.ds(pbase + t * TSZ, TSZ)])
        for j in range(16):
            pltpu.sync_copy(val_v.at[pl.ds(j * 128, 128)],
                            spmem.at[idx_v.at[j]], add=True)
    plsc.subcore_barrier()
    lstep, lg = passes[-1]
    pltpu.sync_copy(spmem.at[pl.ds(((len(passes) - 1) & 1) * SPB + t * TSZ,
                                   TSZ)],
                    a_out.at[lg, lstep * 2 + c, t])


def _sc_build(edge_index):
    mesh = plsc.VectorSubcoreMesh(core_axis_name="c", subcore_axis_name="s")
    f = pl.kernel(
        _sc_build_body,
        mesh=mesh,
        out_type=[
            jax.ShapeDtypeStruct((B, NCH, 16, CHUNK * N // 16), jnp.float32),
            jax.ShapeDtypeStruct((B, N), jnp.float32),
        ],
        scratch_types=[
            pltpu.VMEM((EPT,), jnp.int32),
            pltpu.VMEM((EPT,), jnp.int32),
            pltpu.VMEM((EPT,), jnp.float32),
            pltpu.VMEM((16, 128), jnp.int32),
            pltpu.VMEM((32768,), jnp.float32),
            pltpu.VMEM_SHARED((CHUNK * N,), jnp.float32),
        ],
    )
    A6, deg = f(edge_index)
    return A6.reshape(B, N, N), deg


def _encode_body(x_ref, deg_ref, wp_ref, bp_ref, w1_ref, u1_ref):
    x = x_ref[0]
    h0 = jnp.maximum(x @ wp_ref[...] + bp_ref[0], 0.0)
    dinv = jax.lax.rsqrt(deg_ref[0, 0] + 1.0)
    u1_ref[0] = (h0 * dinv[:, None]) @ w1_ref[...]


def _encode(x, deg, wp, bp, w1):
    return pl.pallas_call(
        _encode_body,
        grid=(B, NRB),
        in_specs=[
            pl.BlockSpec((1, RB, F), lambda g, r: (g, r, 0)),
            pl.BlockSpec((1, 1, RB), lambda g, r: (g, 0, r)),
            pl.BlockSpec((F, H), lambda g, r: (0, 0)),
            pl.BlockSpec((1, H), lambda g, r: (0, 0)),
            pl.BlockSpec((H, H), lambda g, r: (0, 0)),
        ],
        out_specs=pl.BlockSpec((1, RB, H), lambda g, r: (g, r, 0)),
        out_shape=jax.ShapeDtypeStruct((B, N, H), jnp.float32),
    )(x, deg, wp, bp, w1)


def _conv_proj_body(a_ref, u_ref, deg_ref, b_ref, wn_ref, out_ref):
    r = pl.program_id(1)
    u_full = u_ref[0]
    c = a_ref[0] @ u_full
    c = c + u_ref[0, pl.ds(r * RB, RB), :]
    dinv = jax.lax.rsqrt(deg_ref[0, 0, pl.ds(r * RB, RB)] + 1.0)
    h = jnp.maximum(c * dinv[:, None] + b_ref[0], 0.0)
    out_ref[0] = (h * dinv[:, None]) @ wn_ref[...]


def _conv_proj(A, u, deg, b, wn):
    return pl.pallas_call(
        _conv_proj_body,
        grid=(B, NRB),
        in_specs=[
            pl.BlockSpec((1, RB, N), lambda g, r: (g, r, 0)),
            pl.BlockSpec((1, N, H), lambda g, r: (g, 0, 0)),
            pl.BlockSpec((1, 1, N), lambda g, r: (g, 0, 0)),
            pl.BlockSpec((1, H), lambda g, r: (0, 0)),
            pl.BlockSpec((H, H), lambda g, r: (0, 0)),
        ],
        out_specs=pl.BlockSpec((1, RB, H), lambda g, r: (g, r, 0)),
        out_shape=jax.ShapeDtypeStruct((B, N, H), jnp.float32),
    )(A, u, deg, b, wn)


def _conv_pool_body(a_ref, u_ref, deg_ref, b_ref, ge_ref):
    r = pl.program_id(1)
    u_full = u_ref[0]
    c = a_ref[0] @ u_full
    c = c + u_ref[0, pl.ds(r * RB, RB), :]
    dinv = jax.lax.rsqrt(deg_ref[0, 0, pl.ds(r * RB, RB)] + 1.0)
    h = jnp.maximum(c * dinv[:, None] + b_ref[0], 0.0)
    part = jnp.sum(h, axis=0) * (1.0 / N)

    @pl.when(r == 0)
    def _():
        ge_ref[0, 0] = part

    @pl.when(r != 0)
    def _():
        ge_ref[0, 0] = ge_ref[0, 0] + part


def _conv_pool(A, u, deg, b):
    return pl.pallas_call(
        _conv_pool_body,
        grid=(B, NRB),
        in_specs=[
            pl.BlockSpec((1, RB, N), lambda g, r: (g, r, 0)),
            pl.BlockSpec((1, N, H), lambda g, r: (g, 0, 0)),
            pl.BlockSpec((1, 1, N), lambda g, r: (g, 0, 0)),
            pl.BlockSpec((1, H), lambda g, r: (0, 0)),
        ],
        out_specs=pl.BlockSpec((1, 1, H), lambda g, r: (g, 0, 0)),
        out_shape=jax.ShapeDtypeStruct((B, 1, H), jnp.float32),
    )(A, u, deg, b)


def _heads_body(ge_ref, wc_ref, bc_ref, wh_ref, bh_ref, wl_ref, bl_ref,
                wp1_ref, bp1_ref, wp2_ref, bp2_ref, wd_ref, bd_ref,
                ws_ref, bs_ref, o1, o2, o3, o4, o5, o6, o7):
    ge = ge_ref[...]
    o1[...] = ge @ wc_ref[...] + bc_ref[0]
    o2[...] = ge @ wh_ref[...] + bh_ref[0]
    o3[...] = ge @ wl_ref[...] + bl_ref[0]
    o4[...] = ge @ wp1_ref[...] + bp1_ref[0]
    o5[...] = ge @ wp2_ref[...] + bp2_ref[0]
    o6[...] = ge @ wd_ref[...] + bd_ref[0]
    o7[...] = ge @ ws_ref[...] + bs_ref[0]


def _heads(ge, wc, bc, wh, bh, wl, bl, wp1, bp1, wp2, bp2, wd, bd, ws, bs):
    full = lambda a: pl.BlockSpec(a.shape, lambda: tuple(0 for _ in a.shape))
    args = (ge, wc, bc, wh, bh, wl, bl, wp1, bp1, wp2, bp2, wd, bd, ws, bs)
    outs = [
        jax.ShapeDtypeStruct((B, 1), jnp.float32),
        jax.ShapeDtypeStruct((B, 4), jnp.float32),
        jax.ShapeDtypeStruct((B, 3), jnp.float32),
        jax.ShapeDtypeStruct((B, S), jnp.float32),
        jax.ShapeDtypeStruct((B, S), jnp.float32),
        jax.ShapeDtypeStruct((B, S), jnp.float32),
        jax.ShapeDtypeStruct((B, L), jnp.float32),
    ]
    return pl.pallas_call(
        _heads_body,
        in_specs=[full(a) for a in args],
        out_specs=[pl.BlockSpec(o.shape, lambda: tuple(0 for _ in o.shape))
                   for o in outs],
        out_shape=outs,
    )(*args)


def kernel(node_features, edge_index, num_nodes, num_edges, global_features,
           W_proj, b_proj, W_g1, b_g1, W_g2, b_g2, W_critic, b_critic,
           W_high, b_high, W_ltype, b_ltype, W_p1, b_p1, W_p2, b_p2,
           W_deploy, b_deploy, W_select, b_select):
    A, deg = _sc_build(edge_index)
    deg3 = deg.reshape(B, 1, N)
    r2 = lambda v: v.reshape(1, -1)
    u1 = _encode(node_features, deg3, W_proj, r2(b_proj), W_g1)
    u2 = _conv_proj(A, u1, deg3, r2(b_g1), W_g2)
    ge = _conv_pool(A, u2, deg3, r2(b_g2)).reshape(B, H)
    return _heads(ge, W_critic, r2(b_critic), W_high, r2(b_high),
                  W_ltype, r2(b_ltype), W_p1, r2(b_p1), W_p2, r2(b_p2),
                  W_deploy, r2(b_deploy), W_select, r2(b_select))


# speedup vs baseline: 1.0156x; 1.0156x over previous
import functools

import jax
import jax.numpy as jnp
from jax import lax
from jax.experimental import pallas as pl
from jax.experimental.pallas import tpu as pltpu
from jax.experimental.pallas import tpu_sc as plsc

B, N, E, F, H, S, L = 8, 2048, 32768, 256, 256, 2048, 8
RB = 256
NRB = N // RB

CHUNK = 256
NCH = N // CHUNK
SPROWS = CHUNK * (N // 16)
TSL = SPROWS // 16
EPT = E // 16
NCHK = EPT // 16


def _sc_build_body(edges, a_out, deg_out, srcv, dstv, val_v, idx_v, zbuf,
                   spmem):
    c = lax.axis_index("c")
    t = lax.axis_index("s")
    SPB = CHUNK * N
    TSZ = SPB // 16
    DEG0 = 2 * SPB

    def fill(buf, n16, x):
        def w(i, carry):
            buf[pl.ds(i * 16, 16)] = jnp.full((16,), x, jnp.float32)
            return carry
        lax.fori_loop(0, n16, w, 0)

    fill(zbuf, TSZ // 16, 0.0)

    fill(val_v, EPT // 16, 1.0)

    pltpu.sync_copy(zbuf.at[pl.ds(0, 512)],
                    spmem.at[pl.ds(DEG0 + t * 512, 512)])
    plsc.subcore_barrier()
    for gl in range(B // 2):
        g = c * (B // 2) + gl
        pltpu.sync_copy(edges.at[g, 1, pl.ds(t * EPT, EPT)], dstv)

        def dchunk(k, carry):
            d16 = dstv[pl.ds(k * 16, 16)]
            idx_v[pl.ds(k * 16, 16)] = DEG0 + gl * N + d16
            return carry

        lax.fori_loop(0, NCHK, dchunk, 0)
        pltpu.sync_copy(val_v, spmem.at[idx_v], add=True)
    plsc.subcore_barrier()

    @pl.when(t < B // 2)
    def _():
        pltpu.sync_copy(spmem.at[pl.ds(DEG0 + t * N, N)],
                        deg_out.at[c * (B // 2) + t])

    pltpu.sync_copy(zbuf, spmem.at[pl.ds(t * TSZ, TSZ)])
    pltpu.sync_copy(zbuf, spmem.at[pl.ds(SPB + t * TSZ, TSZ)])
    plsc.subcore_barrier()

    passes = [(step, g) for step in range(NCH // 2) for g in range(B)]
    for p, (step, g) in enumerate(passes):
        par = p & 1
        ch = step * 2 + c
        lo = ch * CHUNK
        base = par * SPB
        pltpu.sync_copy(edges.at[g, 0, pl.ds(t * EPT, EPT)], srcv)
        pltpu.sync_copy(edges.at[g, 1, pl.ds(t * EPT, EPT)], dstv)

        def achunk(k, carry):
            s16 = srcv[pl.ds(k * 16, 16)]
            d16 = dstv[pl.ds(k * 16, 16)]
            m = (d16 >= lo) & (d16 < lo + CHUNK)
            fidx = base + (d16 - lo) * N + s16
            idx_v[pl.ds(k * 16, 16)] = jnp.where(m, fidx, base)
            val_v[pl.ds(k * 16, 16)] = jnp.where(m, 1.0, 0.0).astype(
                jnp.float32)
            return carry

        lax.fori_loop(0, NCHK, achunk, 0)
        plsc.subcore_barrier()
        if p > 0:
            pstep, pg = passes[p - 1]
            pch = pstep * 2 + c
            pbase = (1 - par) * SPB
            pltpu.sync_copy(spmem.at[pl.ds(pbase + t * TSZ, TSZ)],
                            a_out.at[pg, pch, t])
            pltpu.sync_copy(zbuf, spmem.at[pl.ds(pbase + t * TSZ, TSZ)])
        pltpu.sync_copy(val_v, spmem.at[idx_v], add=True)
    plsc.subcore_barrier()
    lstep, lg = passes[-1]
    pltpu.sync_copy(spmem.at[pl.ds(((len(passes) - 1) & 1) * SPB + t * TSZ,
                                   TSZ)],
                    a_out.at[lg, lstep * 2 + c, t])


def _sc_build(edge_index):
    mesh = plsc.VectorSubcoreMesh(core_axis_name="c", subcore_axis_name="s")
    f = pl.kernel(
        _sc_build_body,
        mesh=mesh,
        out_type=[
            jax.ShapeDtypeStruct((B, NCH, 16, CHUNK * N // 16), jnp.float32),
            jax.ShapeDtypeStruct((B, N), jnp.float32),
        ],
        scratch_types=[
            pltpu.VMEM((EPT,), jnp.int32),
            pltpu.VMEM((EPT,), jnp.int32),
            pltpu.VMEM((EPT,), jnp.float32),
            pltpu.VMEM((EPT,), jnp.int32),
            pltpu.VMEM((32768,), jnp.float32),
            pltpu.VMEM_SHARED((2 * CHUNK * N + (B // 2) * N,), jnp.float32),
        ],
    )
    A6, deg = f(edge_index)
    return A6.reshape(B, N, N), deg


def _encode_body(x_ref, deg_ref, wp_ref, bp_ref, w1_ref, u1_ref):
    x = x_ref[0]
    h0 = jnp.maximum(x @ wp_ref[...] + bp_ref[0], 0.0)
    dinv = jax.lax.rsqrt(deg_ref[0, 0] + 1.0)
    u1_ref[0] = (h0 * dinv[:, None]) @ w1_ref[...]


def _encode(x, deg, wp, bp, w1):
    return pl.pallas_call(
        _encode_body,
        grid=(B, NRB),
        in_specs=[
            pl.BlockSpec((1, RB, F), lambda g, r: (g, r, 0)),
            pl.BlockSpec((1, 1, RB), lambda g, r: (g, 0, r)),
            pl.BlockSpec((F, H), lambda g, r: (0, 0)),
            pl.BlockSpec((1, H), lambda g, r: (0, 0)),
            pl.BlockSpec((H, H), lambda g, r: (0, 0)),
        ],
        out_specs=pl.BlockSpec((1, RB, H), lambda g, r: (g, r, 0)),
        out_shape=jax.ShapeDtypeStruct((B, N, H), jnp.float32),
    )(x, deg, wp, bp, w1)


def _conv_proj_body(a_ref, u_ref, deg_ref, b_ref, wn_ref, out_ref):
    r = pl.program_id(1)
    u_full = u_ref[0]
    c = a_ref[0] @ u_full
    c = c + u_ref[0, pl.ds(r * RB, RB), :]
    dinv = jax.lax.rsqrt(deg_ref[0, 0, pl.ds(r * RB, RB)] + 1.0)
    h = jnp.maximum(c * dinv[:, None] + b_ref[0], 0.0)
    out_ref[0] = (h * dinv[:, None]) @ wn_ref[...]


def _conv_proj(A, u, deg, b, wn):
    return pl.pallas_call(
        _conv_proj_body,
        grid=(B, NRB),
        in_specs=[
            pl.BlockSpec((1, RB, N), lambda g, r: (g, r, 0)),
            pl.BlockSpec((1, N, H), lambda g, r: (g, 0, 0)),
            pl.BlockSpec((1, 1, N), lambda g, r: (g, 0, 0)),
            pl.BlockSpec((1, H), lambda g, r: (0, 0)),
            pl.BlockSpec((H, H), lambda g, r: (0, 0)),
        ],
        out_specs=pl.BlockSpec((1, RB, H), lambda g, r: (g, r, 0)),
        out_shape=jax.ShapeDtypeStruct((B, N, H), jnp.float32),
    )(A, u, deg, b, wn)


def _conv_pool_body(a_ref, u_ref, deg_ref, b_ref, ge_ref):
    r = pl.program_id(1)
    u_full = u_ref[0]
    c = a_ref[0] @ u_full
    c = c + u_ref[0, pl.ds(r * RB, RB), :]
    dinv = jax.lax.rsqrt(deg_ref[0, 0, pl.ds(r * RB, RB)] + 1.0)
    h = jnp.maximum(c * dinv[:, None] + b_ref[0], 0.0)
    part = jnp.sum(h, axis=0) * (1.0 / N)

    @pl.when(r == 0)
    def _():
        ge_ref[0, 0] = part

    @pl.when(r != 0)
    def _():
        ge_ref[0, 0] = ge_ref[0, 0] + part


def _conv_pool(A, u, deg, b):
    return pl.pallas_call(
        _conv_pool_body,
        grid=(B, NRB),
        in_specs=[
            pl.BlockSpec((1, RB, N), lambda g, r: (g, r, 0)),
            pl.BlockSpec((1, N, H), lambda g, r: (g, 0, 0)),
            pl.BlockSpec((1, 1, N), lambda g, r: (g, 0, 0)),
            pl.BlockSpec((1, H), lambda g, r: (0, 0)),
        ],
        out_specs=pl.BlockSpec((1, 1, H), lambda g, r: (g, 0, 0)),
        out_shape=jax.ShapeDtypeStruct((B, 1, H), jnp.float32),
    )(A, u, deg, b)


def _heads_body(ge_ref, wc_ref, bc_ref, wh_ref, bh_ref, wl_ref, bl_ref,
                wp1_ref, bp1_ref, wp2_ref, bp2_ref, wd_ref, bd_ref,
                ws_ref, bs_ref, o1, o2, o3, o4, o5, o6, o7):
    ge = ge_ref[...]
    o1[...] = ge @ wc_ref[...] + bc_ref[0]
    o2[...] = ge @ wh_ref[...] + bh_ref[0]
    o3[...] = ge @ wl_ref[...] + bl_ref[0]
    o4[...] = ge @ wp1_ref[...] + bp1_ref[0]
    o5[...] = ge @ wp2_ref[...] + bp2_ref[0]
    o6[...] = ge @ wd_ref[...] + bd_ref[0]
    o7[...] = ge @ ws_ref[...] + bs_ref[0]


def _heads(ge, wc, bc, wh, bh, wl, bl, wp1, bp1, wp2, bp2, wd, bd, ws, bs):
    full = lambda a: pl.BlockSpec(a.shape, lambda: tuple(0 for _ in a.shape))
    args = (ge, wc, bc, wh, bh, wl, bl, wp1, bp1, wp2, bp2, wd, bd, ws, bs)
    outs = [
        jax.ShapeDtypeStruct((B, 1), jnp.float32),
        jax.ShapeDtypeStruct((B, 4), jnp.float32),
        jax.ShapeDtypeStruct((B, 3), jnp.float32),
        jax.ShapeDtypeStruct((B, S), jnp.float32),
        jax.ShapeDtypeStruct((B, S), jnp.float32),
        jax.ShapeDtypeStruct((B, S), jnp.float32),
        jax.ShapeDtypeStruct((B, L), jnp.float32),
    ]
    return pl.pallas_call(
        _heads_body,
        in_specs=[full(a) for a in args],
        out_specs=[pl.BlockSpec(o.shape, lambda: tuple(0 for _ in o.shape))
                   for o in outs],
        out_shape=outs,
    )(*args)


def kernel(node_features, edge_index, num_nodes, num_edges, global_features,
           W_proj, b_proj, W_g1, b_g1, W_g2, b_g2, W_critic, b_critic,
           W_high, b_high, W_ltype, b_ltype, W_p1, b_p1, W_p2, b_p2,
           W_deploy, b_deploy, W_select, b_select):
    A, deg = _sc_build(edge_index)
    deg3 = deg.reshape(B, 1, N)
    r2 = lambda v: v.reshape(1, -1)
    u1 = _encode(node_features, deg3, W_proj, r2(b_proj), W_g1)
    u2 = _conv_proj(A, u1, deg3, r2(b_g1), W_g2)
    ge = _conv_pool(A, u2, deg3, r2(b_g2)).reshape(B, H)
    return _heads(ge, W_critic, r2(b_critic), W_high, r2(b_high),
                  W_ltype, r2(b_ltype), W_p1, r2(b_p1), W_p2, r2(b_p2),
                  W_deploy, r2(b_deploy), W_select, r2(b_select))


# speedup vs baseline: 1.0488x; 1.0327x over previous
import functools

import jax
import jax.numpy as jnp
from jax import lax
from jax.experimental import pallas as pl
from jax.experimental.pallas import tpu as pltpu
from jax.experimental.pallas import tpu_sc as plsc

B, N, E, F, H, S, L = 8, 2048, 32768, 256, 256, 2048, 8
RB = 256
NRB = N // RB

CHUNK = 256
SPB = CHUNK * N
TSZ = SPB // 16
NSUB = 4
NBUF = 2
EPT = E // 16
NCHK = EPT // 16
DEG0 = NBUF * SPB
TRASH = DEG0 + (B // 2) * N


def _sc_build_body(edges, a_out, deg_out, srcv, dstv, idx0, idx1, idx2,
                   idx3, didx, val_f, zbuf, spmem):
    c = lax.axis_index("c")
    t = lax.axis_index("s")
    idxs = (idx0, idx1, idx2, idx3)

    def fill(buf, n16, x, dt):
        def w(i, carry):
            buf[pl.ds(i * 16, 16)] = jnp.full((16,), x, dt)
            return carry
        lax.fori_loop(0, n16, w, 0)

    fill(zbuf, TSZ // 16, 0.0, jnp.float32)
    fill(val_f, EPT // 16, 1.0, jnp.float32)

    for j in range(NBUF):
        pltpu.sync_copy(zbuf, spmem.at[pl.ds(j * SPB + t * TSZ, TSZ)])
    pltpu.sync_copy(zbuf.at[pl.ds(0, 1024)],
                    spmem.at[pl.ds(DEG0 + t * 1024, 1024)])
    plsc.subcore_barrier()

    def drain(gd, sd):
        base = ((gd * NSUB + sd) % NBUF) * SPB
        pltpu.sync_copy(spmem.at[pl.ds(base + t * TSZ, TSZ)],
                        a_out.at[gd, c * NSUB + sd, t])
        pltpu.sync_copy(zbuf, spmem.at[pl.ds(base + t * TSZ, TSZ)])

    pend = []
    for g in range(B):
        pltpu.sync_copy(edges.at[g, 0, pl.ds(t * EPT, EPT)], srcv)
        pltpu.sync_copy(edges.at[g, 1, pl.ds(t * EPT, EPT)], dstv)
        degc = DEG0 + (g % (B // 2)) * N

        def achunk(k, carry):
            s16 = srcv[pl.ds(k * 16, 16)]
            d16 = dstv[pl.ds(k * 16, 16)]
            hi = lax.shift_right_logical(d16, 8)
            f = lax.shift_left(d16 & (CHUNK - 1), 11) + s16
            for s in range(NSUB):
                base = ((g * NSUB + s) % NBUF) * SPB
                idxs[s][pl.ds(k * 16, 16)] = jnp.where(
                    hi == c * NSUB + s, base + f, TRASH)
            didx[pl.ds(k * 16, 16)] = degc + d16
            return carry

        lax.fori_loop(0, NCHK, achunk, 0)

        @pl.when(c == g // (B // 2))
        def _():
            pltpu.sync_copy(val_f, spmem.at[didx], add=True)

        for s in range(NSUB):
            pltpu.sync_copy(val_f, spmem.at[idxs[s]], add=True)
            if len(pend) == 1:
                drain(*pend.pop(0))
            plsc.subcore_barrier()
            pend.append((g, s))

    drain(*pend.pop(0))

    @pl.when(t < B // 2)
    def _():
        pltpu.sync_copy(spmem.at[pl.ds(DEG0 + t * N, N)],
                        deg_out.at[c * (B // 2) + t])


def _sc_build(edge_index):
    mesh = plsc.VectorSubcoreMesh(core_axis_name="c", subcore_axis_name="s")
    f = pl.kernel(
        _sc_build_body,
        mesh=mesh,
        out_type=[
            jax.ShapeDtypeStruct((B, 2 * NSUB, 16, TSZ), jnp.float32),
            jax.ShapeDtypeStruct((B, N), jnp.float32),
        ],
        scratch_types=[
            pltpu.VMEM((EPT,), jnp.int32),
            pltpu.VMEM((EPT,), jnp.int32),
            pltpu.VMEM((EPT,), jnp.int32),
            pltpu.VMEM((EPT,), jnp.int32),
            pltpu.VMEM((EPT,), jnp.int32),
            pltpu.VMEM((EPT,), jnp.int32),
            pltpu.VMEM((EPT,), jnp.int32),
            pltpu.VMEM((EPT,), jnp.float32),
            pltpu.VMEM((TSZ,), jnp.float32),
            pltpu.VMEM_SHARED((NBUF * SPB + (B // 2) * N + 16 * 1024,),
                              jnp.float32),
        ],
    )
    A6, deg = f(edge_index)
    return A6.reshape(B, N, N), deg


def _encode_body(x_ref, deg_ref, wp_ref, bp_ref, w1_ref, u1_ref):
    x = x_ref[0]
    h0 = jnp.maximum(x @ wp_ref[...] + bp_ref[0], 0.0)
    dinv = jax.lax.rsqrt(deg_ref[0, 0] + 1.0)
    u1_ref[0] = (h0 * dinv[:, None]) @ w1_ref[...]


def _encode(x, deg, wp, bp, w1):
    return pl.pallas_call(
        _encode_body,
        grid=(B, NRB),
        in_specs=[
            pl.BlockSpec((1, RB, F), lambda g, r: (g, r, 0)),
            pl.BlockSpec((1, 1, RB), lambda g, r: (g, 0, r)),
            pl.BlockSpec((F, H), lambda g, r: (0, 0)),
            pl.BlockSpec((1, H), lambda g, r: (0, 0)),
            pl.BlockSpec((H, H), lambda g, r: (0, 0)),
        ],
        out_specs=pl.BlockSpec((1, RB, H), lambda g, r: (g, r, 0)),
        out_shape=jax.ShapeDtypeStruct((B, N, H), jnp.float32),
    )(x, deg, wp, bp, w1)


def _conv_proj_body(a_ref, u_ref, deg_ref, b_ref, wn_ref, out_ref):
    r = pl.program_id(1)
    u_full = u_ref[0]
    c = a_ref[0].astype(jnp.float32) @ u_full
    c = c + u_ref[0, pl.ds(r * RB, RB), :]
    dinv = jax.lax.rsqrt(deg_ref[0, 0, pl.ds(r * RB, RB)] + 1.0)
    h = jnp.maximum(c * dinv[:, None] + b_ref[0], 0.0)
    out_ref[0] = (h * dinv[:, None]) @ wn_ref[...]


def _conv_proj(A, u, deg, b, wn):
    return pl.pallas_call(
        _conv_proj_body,
        grid=(B, NRB),
        in_specs=[
            pl.BlockSpec((1, RB, N), lambda g, r: (g, r, 0)),
            pl.BlockSpec((1, N, H), lambda g, r: (g, 0, 0)),
            pl.BlockSpec((1, 1, N), lambda g, r: (g, 0, 0)),
            pl.BlockSpec((1, H), lambda g, r: (0, 0)),
            pl.BlockSpec((H, H), lambda g, r: (0, 0)),
        ],
        out_specs=pl.BlockSpec((1, RB, H), lambda g, r: (g, r, 0)),
        out_shape=jax.ShapeDtypeStruct((B, N, H), jnp.float32),
    )(A, u, deg, b, wn)


def _conv_pool_body(a_ref, u_ref, deg_ref, b_ref, ge_ref):
    r = pl.program_id(1)
    u_full = u_ref[0]
    c = a_ref[0].astype(jnp.float32) @ u_full
    c = c + u_ref[0, pl.ds(r * RB, RB), :]
    dinv = jax.lax.rsqrt(deg_ref[0, 0, pl.ds(r * RB, RB)] + 1.0)
    h = jnp.maximum(c * dinv[:, None] + b_ref[0], 0.0)
    part = jnp.sum(h, axis=0) * (1.0 / N)

    @pl.when(r == 0)
    def _():
        ge_ref[0, 0] = part

    @pl.when(r != 0)
    def _():
        ge_ref[0, 0] = ge_ref[0, 0] + part


def _conv_pool(A, u, deg, b):
    return pl.pallas_call(
        _conv_pool_body,
        grid=(B, NRB),
        in_specs=[
            pl.BlockSpec((1, RB, N), lambda g, r: (g, r, 0)),
            pl.BlockSpec((1, N, H), lambda g, r: (g, 0, 0)),
            pl.BlockSpec((1, 1, N), lambda g, r: (g, 0, 0)),
            pl.BlockSpec((1, H), lambda g, r: (0, 0)),
        ],
        out_specs=pl.BlockSpec((1, 1, H), lambda g, r: (g, 0, 0)),
        out_shape=jax.ShapeDtypeStruct((B, 1, H), jnp.float32),
    )(A, u, deg, b)


def _heads_body(ge_ref, wc_ref, bc_ref, wh_ref, bh_ref, wl_ref, bl_ref,
                wp1_ref, bp1_ref, wp2_ref, bp2_ref, wd_ref, bd_ref,
                ws_ref, bs_ref, o1, o2, o3, o4, o5, o6, o7):
    ge = ge_ref[...]
    o1[...] = ge @ wc_ref[...] + bc_ref[0]
    o2[...] = ge @ wh_ref[...] + bh_ref[0]
    o3[...] = ge @ wl_ref[...] + bl_ref[0]
    o4[...] = ge @ wp1_ref[...] + bp1_ref[0]
    o5[...] = ge @ wp2_ref[...] + bp2_ref[0]
    o6[...] = ge @ wd_ref[...] + bd_ref[0]
    o7[...] = ge @ ws_ref[...] + bs_ref[0]


def _heads(ge, wc, bc, wh, bh, wl, bl, wp1, bp1, wp2, bp2, wd, bd, ws, bs):
    full = lambda a: pl.BlockSpec(a.shape, lambda: tuple(0 for _ in a.shape))
    args = (ge, wc, bc, wh, bh, wl, bl, wp1, bp1, wp2, bp2, wd, bd, ws, bs)
    outs = [
        jax.ShapeDtypeStruct((B, 1), jnp.float32),
        jax.ShapeDtypeStruct((B, 4), jnp.float32),
        jax.ShapeDtypeStruct((B, 3), jnp.float32),
        jax.ShapeDtypeStruct((B, S), jnp.float32),
        jax.ShapeDtypeStruct((B, S), jnp.float32),
        jax.ShapeDtypeStruct((B, S), jnp.float32),
        jax.ShapeDtypeStruct((B, L), jnp.float32),
    ]
    return pl.pallas_call(
        _heads_body,
        in_specs=[full(a) for a in args],
        out_specs=[pl.BlockSpec(o.shape, lambda: tuple(0 for _ in o.shape))
                   for o in outs],
        out_shape=outs,
    )(*args)


def kernel(node_features, edge_index, num_nodes, num_edges, global_features,
           W_proj, b_proj, W_g1, b_g1, W_g2, b_g2, W_critic, b_critic,
           W_high, b_high, W_ltype, b_ltype, W_p1, b_p1, W_p2, b_p2,
           W_deploy, b_deploy, W_select, b_select):
    A, deg = _sc_build(edge_index)
    deg3 = deg.reshape(B, 1, N)
    r2 = lambda v: v.reshape(1, -1)
    u1 = _encode(node_features, deg3, W_proj, r2(b_proj), W_g1)
    u2 = _conv_proj(A, u1, deg3, r2(b_g1), W_g2)
    ge = _conv_pool(A, u2, deg3, r2(b_g2)).reshape(B, H)
    return _heads(ge, W_critic, r2(b_critic), W_high, r2(b_high),
                  W_ltype, r2(b_ltype), W_p1, r2(b_p1), W_p2, r2(b_p2),
                  W_deploy, r2(b_deploy), W_select, r2(b_select))


# speedup vs baseline: 1.0818x; 1.0315x over previous
import functools

import jax
import jax.numpy as jnp
from jax import lax
from jax.experimental import pallas as pl
from jax.experimental.pallas import tpu as pltpu
from jax.experimental.pallas import tpu_sc as plsc

B, N, E, F, H, S, L = 8, 2048, 32768, 256, 256, 2048, 8
RB = 256
NRB = N // RB

CHUNK = 256
SPB = CHUNK * N
TSZ = SPB // 16
NSUB = 4
NBUF = 2
EPT = E // 16
NCHK = EPT // 16
DEG0 = NBUF * SPB
TRASH = DEG0 + (B // 2) * N


def _sc_build_body(edges, a_out, deg_out, srcv, dstv, idx0, idx1, idx2,
                   idx3, didx, val_f, zbuf, spmem):
    c = lax.axis_index("c")
    t = lax.axis_index("s")
    idxs = (idx0, idx1, idx2, idx3)

    def fill(buf, n16, x, dt):
        def w(i, carry):
            buf[pl.ds(i * 16, 16)] = jnp.full((16,), x, dt)
            return carry
        lax.fori_loop(0, n16, w, 0)

    fill(zbuf, TSZ // 16, 0.0, jnp.float32)
    fill(val_f, EPT // 16, 1.0, jnp.float32)

    for j in range(NBUF):
        pltpu.sync_copy(zbuf, spmem.at[pl.ds(j * SPB + t * TSZ, TSZ)])
    pltpu.sync_copy(zbuf.at[pl.ds(0, 1024)],
                    spmem.at[pl.ds(DEG0 + t * 1024, 1024)])
    plsc.subcore_barrier()

    def drain(gd, sd):
        base = ((gd * NSUB + sd) % NBUF) * SPB
        pltpu.sync_copy(spmem.at[pl.ds(base + t * TSZ, TSZ)],
                        a_out.at[gd, c * NSUB + sd, t])
        pltpu.sync_copy(zbuf, spmem.at[pl.ds(base + t * TSZ, TSZ)])

    pend = []
    for g in range(B):
        pltpu.sync_copy(edges.at[g, 0, pl.ds(t * EPT, EPT)], srcv)
        pltpu.sync_copy(edges.at[g, 1, pl.ds(t * EPT, EPT)], dstv)
        degc = DEG0 + (g % (B // 2)) * N

        def achunk(k, carry):
            s16 = srcv[pl.ds(k * 16, 16)]
            d16 = dstv[pl.ds(k * 16, 16)]
            hi = lax.shift_right_logical(d16, 8)
            f = lax.shift_left(d16 & (CHUNK - 1), 11) + s16
            for s in range(NSUB):
                base = ((g * NSUB + s) % NBUF) * SPB
                idxs[s][pl.ds(k * 16, 16)] = jnp.where(
                    hi == c * NSUB + s, base + f, TRASH)
            didx[pl.ds(k * 16, 16)] = degc + d16
            return carry

        lax.fori_loop(0, NCHK, achunk, 0)

        @pl.when(c == g // (B // 2))
        def _():
            pltpu.sync_copy(val_f, spmem.at[didx], add=True)

        for s in range(NSUB):
            pltpu.sync_copy(val_f, spmem.at[idxs[s]], add=True)
            if len(pend) == 1:
                drain(*pend.pop(0))
            plsc.subcore_barrier()
            pend.append((g, s))

    drain(*pend.pop(0))

    @pl.when(t < B // 2)
    def _():
        pltpu.sync_copy(spmem.at[pl.ds(DEG0 + t * N, N)],
                        deg_out.at[c * (B // 2) + t])


def _sc_build(edge_index):
    mesh = plsc.VectorSubcoreMesh(core_axis_name="c", subcore_axis_name="s")
    f = pl.kernel(
        _sc_build_body,
        mesh=mesh,
        out_type=[
            jax.ShapeDtypeStruct((B, 2 * NSUB, 16, TSZ), jnp.float32),
            jax.ShapeDtypeStruct((B, N), jnp.float32),
        ],
        scratch_types=[
            pltpu.VMEM((EPT,), jnp.int32),
            pltpu.VMEM((EPT,), jnp.int32),
            pltpu.VMEM((EPT,), jnp.int32),
            pltpu.VMEM((EPT,), jnp.int32),
            pltpu.VMEM((EPT,), jnp.int32),
            pltpu.VMEM((EPT,), jnp.int32),
            pltpu.VMEM((EPT,), jnp.int32),
            pltpu.VMEM((EPT,), jnp.float32),
            pltpu.VMEM((TSZ,), jnp.float32),
            pltpu.VMEM_SHARED((NBUF * SPB + (B // 2) * N + 16 * 1024,),
                              jnp.float32),
        ],
    )
    A6, deg = f(edge_index)
    return A6.reshape(B, N, N), deg


def _encode_body(x_ref, wp_ref, bp_ref, w1_ref, u1_ref):
    x = x_ref[0]
    h0 = jnp.maximum(x @ wp_ref[...] + bp_ref[0], 0.0)
    u1_ref[0] = h0 @ w1_ref[...]


def _encode(x, wp, bp, w1):
    return pl.pallas_call(
        _encode_body,
        grid=(B, NRB),
        in_specs=[
            pl.BlockSpec((1, RB, F), lambda g, r: (g, r, 0)),
            pl.BlockSpec((F, H), lambda g, r: (0, 0)),
            pl.BlockSpec((1, H), lambda g, r: (0, 0)),
            pl.BlockSpec((H, H), lambda g, r: (0, 0)),
        ],
        out_specs=pl.BlockSpec((1, RB, H), lambda g, r: (g, r, 0)),
        out_shape=jax.ShapeDtypeStruct((B, N, H), jnp.float32),
    )(x, wp, bp, w1)


def _conv_proj_body(a_ref, u_ref, deg_ref, b_ref, wn_ref, out_ref):
    r = pl.program_id(1)
    dinv_full = jax.lax.rsqrt(deg_ref[0, 0] + 1.0)
    u_full = u_ref[0] * dinv_full[:, None]
    c = a_ref[0].astype(jnp.float32) @ u_full
    dinv = jax.lax.rsqrt(deg_ref[0, 0, pl.ds(r * RB, RB)] + 1.0)
    c = c + u_ref[0, pl.ds(r * RB, RB), :] * dinv[:, None]
    h = jnp.maximum(c * dinv[:, None] + b_ref[0], 0.0)
    out_ref[0] = (h * dinv[:, None]) @ wn_ref[...]


def _conv_proj(A, u, deg, b, wn):
    return pl.pallas_call(
        _conv_proj_body,
        grid=(B, NRB),
        in_specs=[
            pl.BlockSpec((1, RB, N), lambda g, r: (g, r, 0)),
            pl.BlockSpec((1, N, H), lambda g, r: (g, 0, 0)),
            pl.BlockSpec((1, 1, N), lambda g, r: (g, 0, 0)),
            pl.BlockSpec((1, H), lambda g, r: (0, 0)),
            pl.BlockSpec((H, H), lambda g, r: (0, 0)),
        ],
        out_specs=pl.BlockSpec((1, RB, H), lambda g, r: (g, r, 0)),
        out_shape=jax.ShapeDtypeStruct((B, N, H), jnp.float32),
    )(A, u, deg, b, wn)


def _conv_pool_body(a_ref, u_ref, deg_ref, b_ref, ge_ref):
    r = pl.program_id(1)
    u_full = u_ref[0]
    c = a_ref[0].astype(jnp.float32) @ u_full
    c = c + u_ref[0, pl.ds(r * RB, RB), :]
    dinv = jax.lax.rsqrt(deg_ref[0, 0, pl.ds(r * RB, RB)] + 1.0)
    h = jnp.maximum(c * dinv[:, None] + b_ref[0], 0.0)
    part = jnp.sum(h, axis=0) * (1.0 / N)

    @pl.when(r == 0)
    def _():
        ge_ref[0, 0] = part

    @pl.when(r != 0)
    def _():
        ge_ref[0, 0] = ge_ref[0, 0] + part


def _conv_pool(A, u, deg, b):
    return pl.pallas_call(
        _conv_pool_body,
        grid=(B, NRB),
        in_specs=[
            pl.BlockSpec((1, RB, N), lambda g, r: (g, r, 0)),
            pl.BlockSpec((1, N, H), lambda g, r: (g, 0, 0)),
            pl.BlockSpec((1, 1, N), lambda g, r: (g, 0, 0)),
            pl.BlockSpec((1, H), lambda g, r: (0, 0)),
        ],
        out_specs=pl.BlockSpec((1, 1, H), lambda g, r: (g, 0, 0)),
        out_shape=jax.ShapeDtypeStruct((B, 1, H), jnp.float32),
    )(A, u, deg, b)


def _heads_body(ge_ref, wc_ref, bc_ref, wh_ref, bh_ref, wl_ref, bl_ref,
                wp1_ref, bp1_ref, wp2_ref, bp2_ref, wd_ref, bd_ref,
                ws_ref, bs_ref, o1, o2, o3, o4, o5, o6, o7):
    ge = ge_ref[...]
    o1[...] = ge @ wc_ref[...] + bc_ref[0]
    o2[...] = ge @ wh_ref[...] + bh_ref[0]
    o3[...] = ge @ wl_ref[...] + bl_ref[0]
    o4[...] = ge @ wp1_ref[...] + bp1_ref[0]
    o5[...] = ge @ wp2_ref[...] + bp2_ref[0]
    o6[...] = ge @ wd_ref[...] + bd_ref[0]
    o7[...] = ge @ ws_ref[...] + bs_ref[0]


def _heads(ge, wc, bc, wh, bh, wl, bl, wp1, bp1, wp2, bp2, wd, bd, ws, bs):
    full = lambda a: pl.BlockSpec(a.shape, lambda: tuple(0 for _ in a.shape))
    args = (ge, wc, bc, wh, bh, wl, bl, wp1, bp1, wp2, bp2, wd, bd, ws, bs)
    outs = [
        jax.ShapeDtypeStruct((B, 1), jnp.float32),
        jax.ShapeDtypeStruct((B, 4), jnp.float32),
        jax.ShapeDtypeStruct((B, 3), jnp.float32),
        jax.ShapeDtypeStruct((B, S), jnp.float32),
        jax.ShapeDtypeStruct((B, S), jnp.float32),
        jax.ShapeDtypeStruct((B, S), jnp.float32),
        jax.ShapeDtypeStruct((B, L), jnp.float32),
    ]
    return pl.pallas_call(
        _heads_body,
        in_specs=[full(a) for a in args],
        out_specs=[pl.BlockSpec(o.shape, lambda: tuple(0 for _ in o.shape))
                   for o in outs],
        out_shape=outs,
    )(*args)


def kernel(node_features, edge_index, num_nodes, num_edges, global_features,
           W_proj, b_proj, W_g1, b_g1, W_g2, b_g2, W_critic, b_critic,
           W_high, b_high, W_ltype, b_ltype, W_p1, b_p1, W_p2, b_p2,
           W_deploy, b_deploy, W_select, b_select):
    A, deg = _sc_build(edge_index)
    deg3 = deg.reshape(B, 1, N)
    r2 = lambda v: v.reshape(1, -1)
    u1 = _encode(node_features, W_proj, r2(b_proj), W_g1)
    u2 = _conv_proj(A, u1, deg3, r2(b_g1), W_g2)
    ge = _conv_pool(A, u2, deg3, r2(b_g2)).reshape(B, H)
    return _heads(ge, W_critic, r2(b_critic), W_high, r2(b_high),
                  W_ltype, r2(b_ltype), W_p1, r2(b_p1), W_p2, r2(b_p2),
                  W_deploy, r2(b_deploy), W_select, r2(b_select))
